# baseline (speedup 1.0000x reference)
import functools

import jax
import jax.numpy as jnp
from jax import lax
from jax.experimental import pallas as pl
from jax.experimental.pallas import tpu as pltpu
from jax.experimental.pallas import tpu_sc as plsc

NC = 2
NS = 16
NW = NC * NS
LANES = 16

R = 32


@jax.jit
def _gather2d(x, table):
    n_rows, n_cols = x.shape
    rows_per_w = n_rows // NW
    n = rows_per_w // R
    cw = R * n_cols
    tail = n_cols % LANES
    acols = n_cols - tail
    tw = R * tail
    load_offs = list(range(0, acols, LANES))
    if tail:
        load_offs.append(n_cols - LANES)
    store_offs = list(range(0, acols, LANES))
    mesh = plsc.VectorSubcoreMesh(core_axis_name="c", subcore_axis_name="s")

    @functools.partial(
        pl.kernel,
        out_type=(
            jax.ShapeDtypeStruct((n_rows, n_cols), jnp.float32),
            jax.ShapeDtypeStruct((n_rows * tail,), jnp.float32),
        ),
        mesh=mesh,
        scratch_types=[
            pltpu.VMEM((R, n_cols), jnp.int32),
            pltpu.VMEM((R, n_cols), jnp.int32),
            pltpu.VMEM((cw,), jnp.int32),
            pltpu.VMEM((cw,), jnp.int32),
            pltpu.VMEM((cw + LANES,), jnp.float32),
            pltpu.VMEM((cw + LANES,), jnp.float32),
            pltpu.VMEM((R, n_cols), jnp.float32),
            pltpu.VMEM((R, n_cols), jnp.float32),
            pltpu.VMEM((tw + LANES,), jnp.float32),
            pltpu.VMEM((tw + LANES,), jnp.float32),
            pltpu.SemaphoreType.DMA,
            pltpu.SemaphoreType.DMA,
            pltpu.SemaphoreType.DMA,
            pltpu.SemaphoreType.DMA,
            pltpu.SemaphoreType.DMA,
            pltpu.SemaphoreType.DMA,
        ],
    )
    def k(x_hbm, tab_hbm, out_hbm, tail_hbm,
          xa0, xa1, il0, il1, vl0, vl1, va0, va1, vt0, vt1,
          si0, si1, sg0, sg1, so0, so1):
        wid = lax.axis_index("s") * NC + lax.axis_index("c")
        base_row = wid * rows_per_w
        xa = [xa0, xa1]
        il = [il0, il1]
        vl = [vl0, vl1]
        va = [va0, va1]
        vt = [vt0, vt1]
        sidx = [si0, si1]
        sgat = [sg0, sg1]
        sout = [so0, so1]

        def start_in(c):
            s = c % 2
            return pltpu.async_copy(
                x_hbm.at[pl.ds(base_row + c * R, R), :], xa[s], sidx[s]
            )

        def detile(c):
            s = c % 2

            def row(r, carry):
                for c0 in load_offs:
                    il[s][pl.ds(r * n_cols + c0, LANES)] = xa[s][r, pl.ds(c0, LANES)]
                return carry

            lax.fori_loop(0, R, row, 0)

        def start_gat(c):
            s = c % 2
            return pltpu.async_copy(
                tab_hbm.at[il[s]], vl[s].at[pl.ds(0, cw)], sgat[s]
            )

        def repack(c):
            s = c % 2

            def row(r, carry):
                for c0 in store_offs:
                    va[s][r, pl.ds(c0, LANES)] = vl[s][pl.ds(r * n_cols + c0, LANES)]
                if tail:
                    vt[s][pl.ds(r * tail, LANES)] = vl[s][pl.ds(r * n_cols + acols, LANES)]
                return carry

            lax.fori_loop(0, R, row, 0)

        def start_out(c):
            s = c % 2
            d1 = pltpu.async_copy(
                va[s], out_hbm.at[pl.ds(base_row + c * R, R), :], sout[s]
            )
            if not tail:
                return (d1,)
            d2 = pltpu.async_copy(
                vt[s].at[pl.ds(0, tw)],
                tail_hbm.at[pl.ds((base_row + c * R) * tail, tw)],
                sout[s],
            )
            return (d1, d2)

        def wait_out(ds_):
            for d in ds_:
                d.wait()

        in_d, gat_d, out_d = {}, {}, {}
        in_d[0] = start_in(0)
        if n > 1:
            in_d[1] = start_in(1)
        for c in range(n):
            in_d[c].wait()
            detile(c)
            if c + 2 < n:
                in_d[c + 2] = start_in(c + 2)
            gat_d[c] = start_gat(c)
            if c >= 1:
                gat_d[c - 1].wait()
                if c >= 3:
                    wait_out(out_d[c - 3])
                repack(c - 1)
                out_d[c - 1] = start_out(c - 1)
        gat_d[n - 1].wait()
        if n >= 3:
            wait_out(out_d[n - 3])
        repack(n - 1)
        out_d[n - 1] = start_out(n - 1)
        if n >= 2:
            wait_out(out_d[n - 2])
        wait_out(out_d[n - 1])

    return k(x, table)


def kernel(x, vocab_table):
    n_rows, n_cols = x.shape
    tail = n_cols % LANES
    out, tails = _gather2d(x.astype(jnp.int32), vocab_table)
    if tail:
        out = out.at[:, n_cols - tail:].set(tails.reshape(n_rows, tail))
    return out


# speedup vs baseline: 1.2965x; 1.2965x over previous
import functools

import jax
import jax.numpy as jnp
from jax import lax
from jax.experimental import pallas as pl
from jax.experimental.pallas import tpu as pltpu
from jax.experimental.pallas import tpu_sc as plsc

NC = 2
NS = 16
NW = NC * NS
LANES = 16

R = 32


@jax.jit
def _gather2d(x, table):
    n_rows, n_cols = x.shape
    rows_per_w = n_rows // NW
    n = rows_per_w // R
    cw = R * n_cols
    tail = n_cols % LANES
    acols = n_cols - tail
    ocols = (n_cols + 127) // 128 * 128
    load_offs = list(range(0, acols, LANES))
    if tail:
        load_offs.append(n_cols - LANES)
    mesh = plsc.VectorSubcoreMesh(core_axis_name="c", subcore_axis_name="s")

    @functools.partial(
        pl.kernel,
        out_type=jax.ShapeDtypeStruct((n_rows, ocols), jnp.float32),
        mesh=mesh,
        scratch_types=[
            pltpu.VMEM((R, n_cols), jnp.int32),
            pltpu.VMEM((R, n_cols), jnp.int32),
            pltpu.VMEM((cw,), jnp.int32),
            pltpu.VMEM((cw,), jnp.int32),
            pltpu.VMEM((cw + LANES,), jnp.float32),
            pltpu.VMEM((cw + LANES,), jnp.float32),
            pltpu.VMEM((R, ocols), jnp.float32),
            pltpu.VMEM((R, ocols), jnp.float32),
            pltpu.SemaphoreType.DMA,
            pltpu.SemaphoreType.DMA,
            pltpu.SemaphoreType.DMA,
            pltpu.SemaphoreType.DMA,
            pltpu.SemaphoreType.DMA,
            pltpu.SemaphoreType.DMA,
        ],
    )
    def k(x_hbm, tab_hbm, out_hbm,
          xa0, xa1, il0, il1, vl0, vl1, va0, va1,
          si0, si1, sg0, sg1, so0, so1):
        wid = lax.axis_index("s") * NC + lax.axis_index("c")
        base_row = wid * rows_per_w
        xa = [xa0, xa1]
        il = [il0, il1]
        vl = [vl0, vl1]
        va = [va0, va1]
        sidx = [si0, si1]
        sgat = [sg0, sg1]
        sout = [so0, so1]

        def start_in(c):
            s = c % 2
            return pltpu.async_copy(
                x_hbm.at[pl.ds(base_row + c * R, R), :], xa[s], sidx[s]
            )

        def detile(c):
            s = c % 2

            def row(r, carry):
                for c0 in load_offs:
                    il[s][pl.ds(r * n_cols + c0, LANES)] = xa[s][r, pl.ds(c0, LANES)]
                return carry

            lax.fori_loop(0, R, row, 0)

        def start_gat(c):
            s = c % 2
            return pltpu.async_copy(
                tab_hbm.at[il[s]], vl[s].at[pl.ds(0, cw)], sgat[s]
            )

        def repack(c):
            s = c % 2

            def row(r, carry):
                for c0 in range(0, acols, LANES):
                    va[s][r, pl.ds(c0, LANES)] = vl[s][pl.ds(r * n_cols + c0, LANES)]
                if tail:
                    va[s][r, pl.ds(acols, LANES)] = vl[s][pl.ds(r * n_cols + acols, LANES)]
                return carry

            lax.fori_loop(0, R, row, 0)

        def start_out(c):
            s = c % 2
            return pltpu.async_copy(
                va[s], out_hbm.at[pl.ds(base_row + c * R, R), :], sout[s]
            )

        in_d, gat_d, out_d = {}, {}, {}
        in_d[0] = start_in(0)
        if n > 1:
            in_d[1] = start_in(1)
        for c in range(n):
            in_d[c].wait()
            detile(c)
            if c + 2 < n:
                in_d[c + 2] = start_in(c + 2)
            gat_d[c] = start_gat(c)
            if c >= 1:
                gat_d[c - 1].wait()
                if c >= 3:
                    out_d[c - 3].wait()
                repack(c - 1)
                out_d[c - 1] = start_out(c - 1)
        gat_d[n - 1].wait()
        if n >= 3:
            out_d[n - 3].wait()
        repack(n - 1)
        out_d[n - 1] = start_out(n - 1)
        if n >= 2:
            out_d[n - 2].wait()
        out_d[n - 1].wait()

    return k(x, table)


def kernel(x, vocab_table):
    out = _gather2d(x.astype(jnp.int32), vocab_table)
    return out[:, : x.shape[1]]
